# Initial kernel scaffold; baseline (speedup 1.0000x reference)
import jax
import jax.numpy as jnp
from jax.experimental import pallas as pl


def kernel(regime_id, table):
    raise NotImplementedError("write your pallas kernel here")


# speedup vs baseline: 4.6089x; 4.6089x over previous
import functools

import jax
import jax.numpy as jnp
from jax import lax
from jax.experimental import pallas as pl
from jax.experimental.pallas import tpu as pltpu
from jax.experimental.pallas import tpu_sc as plsc

_ROWS = 16384
_COLS = 200
_D = 16
_B = _ROWS * _COLS
_NC = 2
_NS = 16
_NW = _NC * _NS
_PER_W = _B // _NW
_CHUNK = 1024
_NCHUNK = _PER_W // _CHUNK
_NGRP = _CHUNK // 16


@functools.partial(
    pl.kernel,
    mesh=plsc.VectorSubcoreMesh(core_axis_name="c", subcore_axis_name="s"),
    compiler_params=pltpu.CompilerParams(needs_layout_passes=False),
    out_type=jax.ShapeDtypeStruct((_B * _D,), jnp.float32),
    scratch_types=[
        pltpu.VMEM((64,), jnp.float32),
        pltpu.VMEM((_CHUNK,), jnp.int32),
        pltpu.VMEM((_CHUNK * _D,), jnp.float32),
    ],
)
def _emb_lookup(table_hbm, idx_hbm, out_hbm, tab_v, idx_v, rows_v):
    wid = lax.axis_index("s") * _NC + lax.axis_index("c")
    base0 = wid * _PER_W
    pltpu.sync_copy(table_hbm, tab_v)
    liota = lax.iota(jnp.int32, 16) * _D

    def chunk_body(g, carry):
        base = base0 + g * _CHUNK
        pltpu.sync_copy(idx_hbm.at[pl.ds(base, _CHUNK)], idx_v)

        def grp_body(h, c2):
            idxv = idx_v[pl.ds(h * 16, 16)]
            src = idxv * _D
            dst_base = h * (16 * _D)
            for k in range(_D):
                vals = plsc.load_gather(tab_v, [src + k])
                plsc.store_scatter(rows_v, [liota + (dst_base + k)], vals)
            return c2

        lax.fori_loop(0, _NGRP, grp_body, 0)
        pltpu.sync_copy(
            rows_v, out_hbm.at[pl.ds(base * _D, _CHUNK * _D)])
        return carry

    lax.fori_loop(0, _NCHUNK, chunk_body, 0)


def kernel(regime_id, table):
    idx = regime_id.reshape(_B).astype(jnp.int32)
    out = _emb_lookup(table.reshape(64), idx)
    return out.reshape(_ROWS, _COLS, _D)


# speedup vs baseline: 4.8523x; 1.0528x over previous
import functools

import jax
import jax.numpy as jnp
from jax import lax
from jax.experimental import pallas as pl
from jax.experimental.pallas import tpu as pltpu
from jax.experimental.pallas import tpu_sc as plsc

_ROWS = 16384
_COLS = 200
_D = 16
_B = _ROWS * _COLS
_NC = 2
_NS = 16
_NW = _NC * _NS
_PER_W = _B // _NW
_CHUNK = 2048
_NCHUNK = _PER_W // _CHUNK
_NGRP = _CHUNK // 16
_NPAIR = _NCHUNK // 2


@functools.partial(
    pl.kernel,
    mesh=plsc.VectorSubcoreMesh(core_axis_name="c", subcore_axis_name="s"),
    compiler_params=pltpu.CompilerParams(needs_layout_passes=False),
    out_type=jax.ShapeDtypeStruct((_B * _D,), jnp.float32),
    scratch_types=[
        pltpu.VMEM((64,), jnp.float32),
        pltpu.VMEM((_CHUNK,), jnp.int32),
        pltpu.VMEM((_CHUNK,), jnp.int32),
        pltpu.VMEM((_CHUNK * _D,), jnp.float32),
        pltpu.VMEM((_CHUNK * _D,), jnp.float32),
        pltpu.SemaphoreType.DMA,
        pltpu.SemaphoreType.DMA,
        pltpu.SemaphoreType.DMA,
        pltpu.SemaphoreType.DMA,
    ],
)
def _emb_lookup(table_hbm, idx_hbm, out_hbm,
                tab_v, idx0, idx1, rows0, rows1, si0, si1, so0, so1):
    wid = lax.axis_index("s") * _NC + lax.axis_index("c")
    base0 = wid * _PER_W
    idxs = (idx0, idx1)
    rows = (rows0, rows1)
    sis = (si0, si1)
    sos = (so0, so1)

    pltpu.sync_copy(table_hbm, tab_v)
    liota = lax.iota(jnp.int32, 16) * _D

    def idx_base(g):
        return base0 + lax.min(g, _NCHUNK - 1) * _CHUNK

    pltpu.async_copy(idx_hbm.at[pl.ds(idx_base(0), _CHUNK)], idx0, si0)
    pltpu.async_copy(idx_hbm.at[pl.ds(idx_base(1), _CHUNK)], idx1, si1)

    def expand(idx_v, rows_v):
        def grp_body(h, c2):
            idxv = idx_v[pl.ds(h * 16, 16)]
            src = idxv * _D
            dst_base = h * (16 * _D)
            for k in range(_D):
                vals = plsc.load_gather(tab_v, [src + k])
                plsc.store_scatter(rows_v, [liota + (dst_base + k)], vals)
            return c2
        lax.fori_loop(0, _NGRP, grp_body, 0, unroll=2)

    def pair_body(gg, carry):
        for b in range(2):
            g = gg * 2 + b
            base = base0 + g * _CHUNK
            pltpu.make_async_copy(
                idx_hbm.at[pl.ds(0, _CHUNK)], idxs[b], sis[b]).wait()

            @pl.when(gg > 0)
            def _():
                pltpu.make_async_copy(
                    rows[b], out_hbm.at[pl.ds(0, _CHUNK * _D)],
                    sos[b]).wait()

            expand(idxs[b], rows[b])
            pltpu.async_copy(
                rows[b], out_hbm.at[pl.ds(base * _D, _CHUNK * _D)], sos[b])
            pltpu.async_copy(
                idx_hbm.at[pl.ds(idx_base(g + 2), _CHUNK)], idxs[b], sis[b])
        return carry

    lax.fori_loop(0, _NPAIR, pair_body, 0)

    for b in range(2):
        pltpu.make_async_copy(
            idx_hbm.at[pl.ds(0, _CHUNK)], idxs[b], sis[b]).wait()
        pltpu.make_async_copy(
            rows[b], out_hbm.at[pl.ds(0, _CHUNK * _D)], sos[b]).wait()


def kernel(regime_id, table):
    idx = regime_id.reshape(_B).astype(jnp.int32)
    out = _emb_lookup(table.reshape(64), idx)
    return out.reshape(_ROWS, _COLS, _D)


# speedup vs baseline: 5.8372x; 1.2030x over previous
import functools

import jax
import jax.numpy as jnp
from jax import lax
from jax.experimental import pallas as pl
from jax.experimental.pallas import tpu as pltpu
from jax.experimental.pallas import tpu_sc as plsc

_ROWS = 16384
_COLS = 200
_D = 16
_B = _ROWS * _COLS
_NC = 2
_NS = 16
_NW = _NC * _NS
_PER_W = _B // _NW
_CHUNK = 2048
_NCHUNK = _PER_W // _CHUNK
_NGRP = _CHUNK // 16
_NPAIR = _NCHUNK // 2


@functools.partial(
    pl.kernel,
    mesh=plsc.VectorSubcoreMesh(core_axis_name="c", subcore_axis_name="s"),
    compiler_params=pltpu.CompilerParams(needs_layout_passes=False),
    out_type=jax.ShapeDtypeStruct((_B * _D,), jnp.float32),
    scratch_types=[
        pltpu.VMEM((64,), jnp.float32),
        pltpu.VMEM((_CHUNK,), jnp.int32),
        pltpu.VMEM((_CHUNK,), jnp.int32),
        pltpu.VMEM((_CHUNK * _D,), jnp.float32),
        pltpu.VMEM((_CHUNK * _D,), jnp.float32),
        pltpu.SemaphoreType.DMA,
        pltpu.SemaphoreType.DMA,
        pltpu.SemaphoreType.DMA,
        pltpu.SemaphoreType.DMA,
    ],
)
def _emb_lookup(table_hbm, idx_hbm, out_hbm,
                tab_v, idx0, idx1, rows0, rows1, si0, si1, so0, so1):
    wid = lax.axis_index("s") * _NC + lax.axis_index("c")
    base0 = wid * _PER_W
    idxs = (idx0, idx1)
    rows = (rows0, rows1)
    sis = (si0, si1)
    sos = (so0, so1)

    pltpu.sync_copy(table_hbm, tab_v)

    def idx_base(g):
        return base0 + lax.min(g, _NCHUNK - 1) * _CHUNK

    pltpu.async_copy(idx_hbm.at[pl.ds(idx_base(0), _CHUNK)], idx0, si0)
    pltpu.async_copy(idx_hbm.at[pl.ds(idx_base(1), _CHUNK)], idx1, si1)

    def expand(idx_v, rows_v):
        def grp_body(h, c2):
            offs = idx_v[pl.ds(h * 16, 16)] * _D
            base = h * (16 * _D)
            for k in range(16):
                rows_v[pl.ds(base + k * _D, _D)] = tab_v[pl.ds(offs[k], _D)]
            return c2
        lax.fori_loop(0, _NGRP, grp_body, 0, unroll=2)

    def pair_body(gg, carry):
        for b in range(2):
            g = gg * 2 + b
            base = base0 + g * _CHUNK
            pltpu.make_async_copy(
                idx_hbm.at[pl.ds(0, _CHUNK)], idxs[b], sis[b]).wait()

            @pl.when(gg > 0)
            def _():
                pltpu.make_async_copy(
                    rows[b], out_hbm.at[pl.ds(0, _CHUNK * _D)],
                    sos[b]).wait()

            expand(idxs[b], rows[b])
            pltpu.async_copy(
                rows[b], out_hbm.at[pl.ds(base * _D, _CHUNK * _D)], sos[b])
            pltpu.async_copy(
                idx_hbm.at[pl.ds(idx_base(g + 2), _CHUNK)], idxs[b], sis[b])
        return carry

    lax.fori_loop(0, _NPAIR, pair_body, 0)

    for b in range(2):
        pltpu.make_async_copy(
            idx_hbm.at[pl.ds(0, _CHUNK)], idxs[b], sis[b]).wait()
        pltpu.make_async_copy(
            rows[b], out_hbm.at[pl.ds(0, _CHUNK * _D)], sos[b]).wait()


def kernel(regime_id, table):
    idx = regime_id.reshape(_B).astype(jnp.int32)
    out = _emb_lookup(table.reshape(64), idx)
    return out.reshape(_ROWS, _COLS, _D)


# speedup vs baseline: 120.9599x; 20.7222x over previous
import functools

import jax
import jax.numpy as jnp
from jax import lax
from jax.experimental import pallas as pl
from jax.experimental.pallas import tpu as pltpu
from jax.experimental.pallas import tpu_sc as plsc

_ROWS = 16384
_COLS = 200
_D = 16
_NW = 32
_IW = _ROWS // _NW
_ISEG = 256
_JT = 8
_NUNIT = (_COLS // _JT) * (_IW // _ISEG)

_GDN = lax.GatherDimensionNumbers(
    offset_dims=(), collapsed_slice_dims=(0,), start_index_map=(0,))


def _vgather(src, idx):
    return lax.gather(src, idx[:, None], _GDN, slice_sizes=(1,),
                      mode=lax.GatherScatterMode.PROMISE_IN_BOUNDS)


@functools.partial(
    pl.kernel,
    mesh=plsc.VectorSubcoreMesh(core_axis_name="c", subcore_axis_name="s"),
    compiler_params=pltpu.CompilerParams(needs_layout_passes=False),
    out_type=jax.ShapeDtypeStruct((_COLS, _D, _ROWS), jnp.float32),
    scratch_types=[
        pltpu.VMEM((256,), jnp.float32),
        pltpu.VMEM((_JT, _ISEG), jnp.int32),
        pltpu.VMEM((_JT, _ISEG), jnp.int32),
        pltpu.VMEM((_JT, _D, _ISEG), jnp.float32),
        pltpu.VMEM((_JT, _D, _ISEG), jnp.float32),
        pltpu.SemaphoreType.DMA,
        pltpu.SemaphoreType.DMA,
        pltpu.SemaphoreType.DMA,
        pltpu.SemaphoreType.DMA,
    ],
)
def _emb_lookup(tab_hbm, idxt_hbm, out_hbm,
                tab_v, idx0, idx1, blk0, blk1, si0, si1, so0, so1):
    wid = lax.axis_index("s") * 2 + lax.axis_index("c")
    i_lo = wid * _IW
    idxs = (idx0, idx1)
    blks = (blk0, blk1)
    sis = (si0, si1)
    sos = (so0, so1)

    pltpu.sync_copy(tab_hbm, tab_v)
    tcol = [tab_v[pl.ds(k * 16, 16)] for k in range(_D)]

    def unit_slices(u):
        uc = lax.min(u, _NUNIT - 1)
        jt = uc // 2
        i0 = i_lo + (uc % 2) * _ISEG
        return pl.ds(jt * _JT, _JT), pl.ds(i0, _ISEG)

    def idx_fetch(u, b):
        js, is_ = unit_slices(u)
        pltpu.async_copy(idxt_hbm.at[js, is_], idxs[b], sis[b])

    idx_fetch(0, 0)
    idx_fetch(1, 1)

    def unit_body(u, carry):
        for b in range(2):
            uu = u * 2 + b
            js, is_ = unit_slices(uu)
            pltpu.make_async_copy(
                idxt_hbm.at[js, is_], idxs[b], sis[b]).wait()

            @pl.when(u > 0)
            def _():
                pltpu.make_async_copy(
                    blks[b], out_hbm.at[js, :, is_], sos[b]).wait()

            idx_v, blk = idxs[b], blks[b]

            def j_body(j, c1):
                def iv_body(iv, c2):
                    idxv = idx_v[j, pl.ds(iv * 16, 16)]
                    for k in range(_D):
                        blk[j, k, pl.ds(iv * 16, 16)] = _vgather(
                            tcol[k], idxv)
                    return c2
                lax.fori_loop(0, _ISEG // 16, iv_body, 0, unroll=2)
                return c1
            lax.fori_loop(0, _JT, j_body, 0)

            pltpu.async_copy(blk, out_hbm.at[js, :, is_], sos[b])
            idx_fetch(uu + 2, b)
        return carry

    lax.fori_loop(0, _NUNIT // 2, unit_body, 0)

    for b in range(2):
        js, is_ = unit_slices(_NUNIT - 2 + b)
        pltpu.make_async_copy(
            idxt_hbm.at[js, is_], idxs[b], sis[b]).wait()
        pltpu.make_async_copy(
            blks[b], out_hbm.at[js, :, is_], sos[b]).wait()


def kernel(regime_id, table):
    idx_t = jnp.transpose(regime_id).astype(jnp.int32)
    tab_t = jnp.pad(jnp.transpose(table), ((0, 0), (0, 12)))
    out_t = _emb_lookup(tab_t.reshape(256), idx_t)
    return jnp.transpose(out_t, (2, 0, 1))
